# baseline (speedup 1.0000x reference)
import functools

import jax
import jax.numpy as jnp
from jax import lax
from jax.experimental import pallas as pl
from jax.experimental.pallas import tpu as pltpu
from jax.experimental.pallas import tpu_sc as plsc

HIDDEN = 128
NUM_EMB = 4
N_EDGES = 320000

_INFO = plsc.get_sparse_core_info()
_NC, _NS = _INFO.num_cores, _INFO.num_subcores
_NW = _NC * _NS
_CHUNK = 80
_GPB = 5
_BLOCK = _CHUNK * _GPB
_PER_W = N_EDGES // (_NW * _BLOCK)
_NBUF = 2


def _sc_lookup(x2_hbm, table_hbm, out_hbm, idx_all, rows, table_spm, sem_g, sem_w):
    sub = lax.axis_index("s")
    wid = sub * _NC + lax.axis_index("c")

    @pl.when(sub == 0)
    def _stage_table():
        pltpu.sync_copy(table_hbm, table_spm)

    pltpu.sync_copy(x2_hbm.at[wid], idx_all)
    plsc.subcore_barrier()

    def gather(g, b, h):
        return pltpu.make_async_copy(
            table_spm.at[idx_all.at[g * _GPB + b]],
            rows.at[h].at[pl.ds(b * _CHUNK, _CHUNK)], sem_g.at[h])

    def write(g, h):
        return pltpu.make_async_copy(
            rows.at[h], out_hbm.at[pl.ds((wid * _PER_W + g) * _BLOCK, _BLOCK)],
            sem_w.at[h])

    for g in range(_PER_W):
        h = g % _NBUF
        if g >= _NBUF:
            write(g - _NBUF, h).wait()
        for b in range(_GPB):
            gather(g, b, h).start()
        for b in range(_GPB):
            gather(g, b, h).wait()
        write(g, h).start()
    for g in range(_PER_W - _NBUF, _PER_W):
        write(g, g % _NBUF).wait()


def kernel(x, table):
    x2 = x.reshape(_NW, _PER_W * _GPB, _CHUNK)
    mesh = plsc.VectorSubcoreMesh(core_axis_name="c", subcore_axis_name="s")
    fn = functools.partial(
        pl.kernel,
        mesh=mesh,
        out_type=jax.ShapeDtypeStruct((N_EDGES, HIDDEN), jnp.float32),
        scratch_types=[
            pltpu.VMEM((_PER_W * _GPB, _CHUNK), jnp.int32),
            pltpu.VMEM((_NBUF, _BLOCK, HIDDEN), jnp.float32),
            pltpu.VMEM_SHARED((NUM_EMB, HIDDEN), jnp.float32),
            pltpu.SemaphoreType.DMA((_NBUF,)),
            pltpu.SemaphoreType.DMA((_NBUF,)),
        ],
    )(_sc_lookup)
    return fn(x2, table)


# speedup vs baseline: 1.0423x; 1.0423x over previous
import functools

import jax
import jax.numpy as jnp
from jax import lax
from jax.experimental import pallas as pl
from jax.experimental.pallas import tpu as pltpu
from jax.experimental.pallas import tpu_sc as plsc

HIDDEN = 128
NUM_EMB = 4
N_EDGES = 320000

_INFO = plsc.get_sparse_core_info()
_NC, _NS = _INFO.num_cores, _INFO.num_subcores
_NW = _NC * _NS
_CHUNK = 80
_PER_W = N_EDGES // (_NW * _CHUNK)
_NBUF = 6


def _sc_lookup(x2_hbm, table_hbm, out_hbm, idx_all, rows, table_spm, sem_g, sem_w):
    sub = lax.axis_index("s")
    wid = sub * _NC + lax.axis_index("c")

    @pl.when(sub == 0)
    def _stage_table():
        pltpu.sync_copy(table_hbm, table_spm)

    pltpu.sync_copy(x2_hbm.at[wid], idx_all)
    plsc.subcore_barrier()

    def gather(c, h):
        return pltpu.make_async_copy(
            table_spm.at[idx_all.at[c]], rows.at[h], sem_g.at[h])

    def write(c, h):
        return pltpu.make_async_copy(
            rows.at[h], out_hbm.at[pl.ds((wid * _PER_W + c) * _CHUNK, _CHUNK)],
            sem_w.at[h])

    gather(0, 0).start()
    for c in range(_PER_W):
        h = c % _NBUF
        if c + 1 < _PER_W:
            hn = (c + 1) % _NBUF
            if c + 1 >= _NBUF:
                write(c + 1 - _NBUF, hn).wait()
            gather(c + 1, hn).start()
        gather(c, h).wait()
        write(c, h).start()
    for c in range(_PER_W - _NBUF, _PER_W):
        write(c, c % _NBUF).wait()


def kernel(x, table):
    x2 = x.reshape(_NW, _PER_W, _CHUNK)
    mesh = plsc.VectorSubcoreMesh(core_axis_name="c", subcore_axis_name="s")
    fn = functools.partial(
        pl.kernel,
        mesh=mesh,
        out_type=jax.ShapeDtypeStruct((N_EDGES, HIDDEN), jnp.float32),
        scratch_types=[
            pltpu.VMEM((_PER_W, _CHUNK), jnp.int32),
            pltpu.VMEM((_NBUF, _CHUNK, HIDDEN), jnp.float32),
            pltpu.VMEM_SHARED((NUM_EMB, HIDDEN), jnp.float32),
            pltpu.SemaphoreType.DMA((_NBUF,)),
            pltpu.SemaphoreType.DMA((_NBUF,)),
        ],
    )(_sc_lookup)
    return fn(x2, table)
